# baseline (speedup 1.0000x reference)
import functools

import jax
import jax.numpy as jnp
from jax import lax
from jax.experimental import pallas as pl
from jax.experimental.pallas import tpu as pltpu
from jax.experimental.pallas import tpu_sc as plsc

_NUM_TOKENS = 32
_DIM = 768
_BATCH = 16384
_LANES = 16

_INFO = plsc.get_sparse_core_info()
_NC = _INFO.num_cores
_NS = _INFO.num_subcores
_NW = _NC * _NS
_BPW = _BATCH // _NW
_CHUNK = 64
_NCHUNKS = _BPW // _CHUNK
_GPC = _CHUNK // _LANES
_NGROUPS = _BPW // _LANES
_CSTEPS = _DIM // _LANES


def _body(tokens_hbm, idx_hbm, out_hbm, table_v, idx_v, rows_v, s0, s1):
    wid = lax.axis_index("s") * _NC + lax.axis_index("c")
    base = wid * _BPW

    pltpu.sync_copy(tokens_hbm, table_v)
    pltpu.sync_copy(idx_hbm.at[pl.ds(base, _BPW)], idx_v)

    col_iota = lax.iota(jnp.int32, _LANES)

    def group(g, _):
        chunk = g // _GPC
        parity = lax.rem(chunk, 2)
        buf_base = parity * _CHUNK

        @pl.when(jnp.logical_and(lax.rem(g, _GPC) == 0, chunk >= 2))
        def _():
            @pl.when(parity == 0)
            def _():
                pltpu.make_async_copy(
                    rows_v.at[pl.ds(0, _CHUNK)],
                    out_hbm.at[pl.ds(0, _CHUNK)], s0).wait()

            @pl.when(parity == 1)
            def _():
                pltpu.make_async_copy(
                    rows_v.at[pl.ds(0, _CHUNK)],
                    out_hbm.at[pl.ds(0, _CHUNK)], s1).wait()

        idxv = idx_v[pl.ds(g * _LANES, _LANES)]
        for r in range(_LANES):
            tok = idxv[r]
            buf_row = buf_base + lax.rem(g, _GPC) * _LANES + r
            for c in range(_CSTEPS):
                rows_v[buf_row, pl.ds(c * _LANES, _LANES)] = (
                    table_v[tok, pl.ds(c * _LANES, _LANES)])

        @pl.when(lax.rem(g, _GPC) == _GPC - 1)
        def _():
            @pl.when(parity == 0)
            def _():
                pltpu.async_copy(
                    rows_v.at[pl.ds(0, _CHUNK)],
                    out_hbm.at[pl.ds(base + chunk * _CHUNK, _CHUNK)], s0)

            @pl.when(parity == 1)
            def _():
                pltpu.async_copy(
                    rows_v.at[pl.ds(_CHUNK, _CHUNK)],
                    out_hbm.at[pl.ds(base + chunk * _CHUNK, _CHUNK)], s1)

        return _

    lax.fori_loop(0, _NGROUPS, group, None)

    pltpu.make_async_copy(
        rows_v.at[pl.ds(0, _CHUNK)], out_hbm.at[pl.ds(0, _CHUNK)], s0).wait()
    pltpu.make_async_copy(
        rows_v.at[pl.ds(0, _CHUNK)], out_hbm.at[pl.ds(0, _CHUNK)], s1).wait()


_lookup = functools.partial(
    pl.kernel,
    out_type=jax.ShapeDtypeStruct((_BATCH, _DIM), jnp.float32),
    mesh=plsc.VectorSubcoreMesh(core_axis_name="c", subcore_axis_name="s"),
    scratch_types=[
        pltpu.VMEM((_NUM_TOKENS, _DIM), jnp.float32),
        pltpu.VMEM((_BPW,), jnp.int32),
        pltpu.VMEM((2 * _CHUNK, _DIM), jnp.float32),
        pltpu.SemaphoreType.DMA,
        pltpu.SemaphoreType.DMA,
    ],
)(_body)


@jax.jit
def kernel(tokens, indices):
    return _lookup(tokens, indices)


# speedup vs baseline: 3.0853x; 3.0853x over previous
import functools

import jax
import jax.numpy as jnp
from jax import lax
from jax.experimental import pallas as pl
from jax.experimental.pallas import tpu as pltpu
from jax.experimental.pallas import tpu_sc as plsc

_NUM_TOKENS = 32
_DIM = 768
_BATCH = 16384
_LANES = 16

_INFO = plsc.get_sparse_core_info()
_NC = _INFO.num_cores
_NS = _INFO.num_subcores
_NW = _NC * _NS
_BPW = _BATCH // _NW
_NGROUPS = _BPW // _LANES
_LAG = 8


def _body(tokens_hbm, idx_hbm, out_hbm, table_v, idx_v, sem):
    wid = lax.axis_index("s") * _NC + lax.axis_index("c")
    base = wid * _BPW

    pltpu.sync_copy(tokens_hbm, table_v)
    pltpu.sync_copy(idx_hbm.at[pl.ds(base, _BPW)], idx_v)

    def group(g, _):
        idxv = idx_v[pl.ds(g * _LANES, _LANES)]
        for r in range(_LANES):
            tok = idxv[r]
            pltpu.async_copy(
                table_v.at[tok], out_hbm.at[base + g * _LANES + r], sem)

        @pl.when(g >= _LAG)
        def _():
            pltpu.make_async_copy(
                table_v.at[pl.ds(0, _LANES)],
                out_hbm.at[pl.ds(0, _LANES)], sem).wait()

        return _

    lax.fori_loop(0, _NGROUPS, group, None)

    def drain(i, _):
        pltpu.make_async_copy(
            table_v.at[pl.ds(0, _LANES)],
            out_hbm.at[pl.ds(0, _LANES)], sem).wait()
        return _

    lax.fori_loop(0, _LAG, drain, None)


_lookup = functools.partial(
    pl.kernel,
    out_type=jax.ShapeDtypeStruct((_BATCH, _DIM), jnp.float32),
    mesh=plsc.VectorSubcoreMesh(core_axis_name="c", subcore_axis_name="s"),
    scratch_types=[
        pltpu.VMEM((_NUM_TOKENS, _DIM), jnp.float32),
        pltpu.VMEM((_BPW,), jnp.int32),
        pltpu.SemaphoreType.DMA,
    ],
)(_body)


@jax.jit
def kernel(tokens, indices):
    return _lookup(tokens, indices)
